# baseline (speedup 1.0000x reference)
import functools

import jax
import jax.numpy as jnp
from jax import lax
from jax.experimental import pallas as pl
from jax.experimental.pallas import tpu as pltpu
from jax.experimental.pallas import tpu_sc as plsc

EMBED_DIM = 32
BATCH = 4096
NUM_ROWS = 100002

_NUM_CORES = 2
_NUM_SUBCORES = 16
_NUM_WORKERS = _NUM_CORES * _NUM_SUBCORES
_LANES = 16

_mesh = plsc.VectorSubcoreMesh(core_axis_name="c", subcore_axis_name="s")


@functools.partial(
    pl.kernel,
    mesh=_mesh,
    out_type=jax.ShapeDtypeStruct((EMBED_DIM, BATCH), jnp.float32),
    scratch_types=[
        pltpu.VMEM((NUM_ROWS,), jnp.float32),
        pltpu.VMEM((BATCH,), jnp.int32),
        pltpu.VMEM((BATCH,), jnp.float32),
        pltpu.SemaphoreType.DMA,
    ],
    compiler_params=pltpu.CompilerParams(needs_layout_passes=False),
)
def _sc_gather_t(idx_hbm, table_t_hbm, out_t_hbm, trow_v, idx_v, col_v, sem):
    dim = lax.axis_index("s") * _NUM_CORES + lax.axis_index("c")
    row_cp = pltpu.async_copy(table_t_hbm.at[dim], trow_v, sem)
    idx_cp = pltpu.async_copy(idx_hbm.at[:], idx_v, sem)
    row_cp.wait()
    idx_cp.wait()

    @plsc.parallel_loop(0, BATCH, step=_LANES, unroll=16)
    def gather_group(base):
        iv = idx_v[pl.ds(base, _LANES)]
        col_v[pl.ds(base, _LANES)] = plsc.load_gather(trow_v, [iv])
    pltpu.sync_copy(col_v, out_t_hbm.at[dim])


def kernel(id_idx, table):
    out_t = _sc_gather_t(id_idx.astype(jnp.int32), table.T)
    return out_t.T


# speedup vs baseline: 1.0052x; 1.0052x over previous
import functools

import jax
import jax.numpy as jnp
from jax import lax
from jax.experimental import pallas as pl
from jax.experimental.pallas import tpu as pltpu
from jax.experimental.pallas import tpu_sc as plsc

EMBED_DIM = 32
BATCH = 4096
NUM_ROWS = 100002

_NUM_CORES = 2
_NUM_SUBCORES = 16
_NUM_WORKERS = _NUM_CORES * _NUM_SUBCORES
_LANES = 16

_mesh = plsc.VectorSubcoreMesh(core_axis_name="c", subcore_axis_name="s")


@functools.partial(
    pl.kernel,
    mesh=_mesh,
    out_type=jax.ShapeDtypeStruct((EMBED_DIM, BATCH), jnp.float32),
    scratch_types=[
        pltpu.VMEM((NUM_ROWS,), jnp.float32),
        pltpu.VMEM((BATCH,), jnp.int32),
        pltpu.VMEM((BATCH,), jnp.float32),
        pltpu.SemaphoreType.DMA,
    ],
    compiler_params=pltpu.CompilerParams(needs_layout_passes=False),
)
def _sc_gather_t(idx_hbm, table_t_hbm, out_t_hbm, trow_v, idx_v, col_v, sem):
    dim = lax.axis_index("s") * _NUM_CORES + lax.axis_index("c")
    row_cp = pltpu.async_copy(table_t_hbm.at[dim], trow_v, sem)
    idx_cp = pltpu.async_copy(idx_hbm.at[:], idx_v, sem)
    row_cp.wait()
    idx_cp.wait()

    @plsc.parallel_loop(0, BATCH, step=_LANES, unroll=8)
    def gather_group(base):
        iv = idx_v[pl.ds(base, _LANES)]
        col_v[pl.ds(base, _LANES)] = plsc.load_gather(trow_v, [iv])
    pltpu.sync_copy(col_v, out_t_hbm.at[dim])


def kernel(id_idx, table):
    out_t = _sc_gather_t(id_idx.astype(jnp.int32), table.T)
    return out_t.T
